# baseline (speedup 1.0000x reference)
import functools

import jax
import jax.numpy as jnp
from jax.experimental import pallas as pl
from jax.experimental.pallas import tpu as pltpu
from jax.experimental.pallas import tpu_sc as plsc

_SHAPE = (2, 16384, 1024)
_BLOCK_ROWS = 512
_TILE = (8, 128)


def _copy_kernel(in_ref, out_ref):
    out_ref[...] = in_ref[...]


def _tc_copy(cache_next):
    grid = (_SHAPE[1] // _BLOCK_ROWS,)
    block = (2, _BLOCK_ROWS, _SHAPE[2])
    return pl.pallas_call(
        _copy_kernel,
        grid=grid,
        out_shape=jax.ShapeDtypeStruct(_SHAPE, jnp.float32),
        in_specs=[pl.BlockSpec(block, lambda i: (0, i, 0))],
        out_specs=pl.BlockSpec(block, lambda i: (0, i, 0)),
    )(cache_next)


_sc_mesh = plsc.VectorSubcoreMesh(core_axis_name="c", subcore_axis_name="s")


@functools.partial(
    pl.kernel,
    mesh=_sc_mesh,
    out_type=jax.ShapeDtypeStruct(_TILE, jnp.float32),
    scratch_types=[
        pltpu.VMEM(_TILE, jnp.float32),
        pltpu.VMEM((16,), jnp.float32),
    ],
)
def _sc_rmw(key_hbm, cache_hbm, patch_hbm, tile_v, key_v):
    cid = jax.lax.axis_index("c")
    sid = jax.lax.axis_index("s")

    @pl.when((cid == 0) & (sid == 0))
    def _():
        pltpu.sync_copy(key_hbm, key_v)
        pltpu.sync_copy(
            cache_hbm.at[1, pl.ds(0, _TILE[0]), pl.ds(0, _TILE[1])], tile_v
        )
        lane = jax.lax.iota(jnp.int32, 16)
        chunk = tile_v[0, pl.ds(0, 16)]
        tile_v[0, pl.ds(0, 16)] = chunk + jnp.where(
            lane == 1, 2.0 * key_v[...], 0.0
        )
        pltpu.sync_copy(tile_v, patch_hbm)


def _stitch_kernel(in_ref, patch_ref, out_ref):
    del in_ref
    out_ref[...] = patch_ref[...].reshape(1, *_TILE)


def _tc_stitch(copied, patch):
    block = (1, *_TILE)
    return pl.pallas_call(
        _stitch_kernel,
        grid=(1,),
        out_shape=jax.ShapeDtypeStruct(_SHAPE, jnp.float32),
        in_specs=[
            pl.BlockSpec(block, lambda i: (1, 0, 0)),
            pl.BlockSpec(_TILE, lambda i: (0, 0)),
        ],
        out_specs=pl.BlockSpec(block, lambda i: (1, 0, 0)),
        input_output_aliases={0: 0},
    )(copied, patch)


def kernel(key, cache_next):
    key16 = jnp.broadcast_to(key, (16,))
    copied = _tc_copy(cache_next)
    patch = _sc_rmw(key16, cache_next)
    out = _tc_stitch(copied, patch)
    return key, out


# speedup vs baseline: 1.2182x; 1.2182x over previous
import jax
import jax.numpy as jnp
from jax.experimental import pallas as pl
from jax.experimental.pallas import tpu as pltpu

_SHAPE = (2, 16384, 1024)
_BLOCK_ROWS = 1024


def _copy_update_kernel(key_ref, in_ref, out_ref):
    out_ref[...] = in_ref[...]

    @pl.when(pl.program_id(0) == 0)
    def _():
        row = jax.lax.broadcasted_iota(jnp.int32, (8, 128), 0)
        col = jax.lax.broadcasted_iota(jnp.int32, (8, 128), 1)
        mask = (row == 0) & (col == 1)
        out_ref[1, 0:8, 0:128] = in_ref[1, 0:8, 0:128] + jnp.where(
            mask, 2.0 * key_ref[0], 0.0
        )


def kernel(key, cache_next):
    grid = (_SHAPE[1] // _BLOCK_ROWS,)
    block = (2, _BLOCK_ROWS, _SHAPE[2])
    out = pl.pallas_call(
        _copy_update_kernel,
        grid=grid,
        out_shape=jax.ShapeDtypeStruct(_SHAPE, jnp.float32),
        in_specs=[
            pl.BlockSpec(memory_space=pltpu.SMEM),
            pl.BlockSpec(block, lambda i: (0, i, 0)),
        ],
        out_specs=pl.BlockSpec(block, lambda i: (0, i, 0)),
    )(key, cache_next)
    return key, out
